# Initial kernel scaffold; baseline (speedup 1.0000x reference)
import jax
import jax.numpy as jnp
from jax.experimental import pallas as pl


def kernel(input_image, delta_x, delta_y):
    raise NotImplementedError("write your pallas kernel here")


# speedup vs baseline: 2.7456x; 2.7456x over previous
import functools

import jax
import jax.numpy as jnp
from jax import lax
from jax.experimental import pallas as pl
from jax.experimental.pallas import tpu as pltpu
from jax.experimental.pallas import tpu_sc as plsc

B, C, H, W = 4, 192, 224, 224
HW = H * W
L = 16
ROWS_PER_BLK = 8
BLK = ROWS_PER_BLK * W
NBLK = H // ROWS_PER_BLK
VPR = W // L
NUNITS = B * C
NWORKERS = 32
UPW = NUNITS // NWORKERS
SX = W / (W - 1.0)
SY = H / (H - 1.0)


def _floor_to_int(v):
    t = v.astype(jnp.int32)
    tf = t.astype(jnp.float32)
    ix0 = t - jnp.where(tf > v, 1, 0).astype(jnp.int32)
    return ix0, ix0.astype(jnp.float32)


def _warp_body(img_hbm, dx_hbm, dy_hbm, out_hbm, imgbuf, dxbuf, dybuf, outbuf, xsbuf):
    wid = lax.axis_index("s") * 2 + lax.axis_index("c")

    def init_xs(i, _):
        xv = lax.iota(jnp.int32, L) + i * L
        xsbuf[pl.ds(i * L, L)] = xv.astype(jnp.float32) * SX - 0.5
        return 0
    lax.fori_loop(0, VPR, init_xs, 0)

    def unit_body(u, _):
        unit = wid * UPW + u
        b = unit // C
        pltpu.sync_copy(img_hbm.at[unit], imgbuf)

        def blk_body(blk, _):
            off = blk * BLK
            pltpu.sync_copy(dx_hbm.at[b, pl.ds(off, BLK)], dxbuf)
            pltpu.sync_copy(dy_hbm.at[b, pl.ds(off, BLK)], dybuf)

            def row_body(r, _):
                y = blk * ROWS_PER_BLK + r
                ys = y.astype(jnp.float32) * SY - 0.5

                def vec_body(v, _):
                    lv = (r * VPR + v) * L
                    dxv = dxbuf[pl.ds(lv, L)]
                    dyv = dybuf[pl.ds(lv, L)]
                    ix = xsbuf[pl.ds(v * L, L)] + dxv * SX
                    iy = jnp.full((L,), ys, jnp.float32) + dyv * SY

                    ix0, ix0f = _floor_to_int(ix)
                    iy0, iy0f = _floor_to_int(iy)
                    fx = ix - ix0f
                    fy = iy - iy0f

                    ix0c = jnp.minimum(jnp.maximum(ix0, 0), W - 1)
                    ix1c = jnp.minimum(jnp.maximum(ix0 + 1, 0), W - 1)
                    iy0c = jnp.minimum(jnp.maximum(iy0, 0), H - 1)
                    iy1c = jnp.minimum(jnp.maximum(iy0 + 1, 0), H - 1)

                    zero = jnp.zeros((L,), jnp.float32)
                    vx0 = (ix0f >= 0.0) & (ix0f <= W - 1.0)
                    vx1 = (ix0f >= -1.0) & (ix0f <= W - 2.0)
                    vy0 = (iy0f >= 0.0) & (iy0f <= H - 1.0)
                    vy1 = (iy0f >= -1.0) & (iy0f <= H - 2.0)
                    wx0 = jnp.where(vx0, 1.0 - fx, zero)
                    wx1 = jnp.where(vx1, fx, zero)
                    wy0 = jnp.where(vy0, 1.0 - fy, zero)
                    wy1 = jnp.where(vy1, fy, zero)

                    row0 = iy0c * W
                    row1 = iy1c * W
                    v00 = plsc.load_gather(imgbuf, [row0 + ix0c])
                    v01 = plsc.load_gather(imgbuf, [row0 + ix1c])
                    v10 = plsc.load_gather(imgbuf, [row1 + ix0c])
                    v11 = plsc.load_gather(imgbuf, [row1 + ix1c])

                    acc = (v00 * wx0 + v01 * wx1) * wy0 + (v10 * wx0 + v11 * wx1) * wy1
                    outbuf[pl.ds(lv, L)] = acc
                    return 0
                lax.fori_loop(0, VPR, vec_body, 0)
                return 0
            lax.fori_loop(0, ROWS_PER_BLK, row_body, 0)
            pltpu.sync_copy(outbuf, out_hbm.at[unit, pl.ds(off, BLK)])
            return 0
        lax.fori_loop(0, NBLK, blk_body, 0)
        return 0
    lax.fori_loop(0, UPW, unit_body, 0)


@jax.jit
def _warp(img2, dxr, dyr):
    mesh = plsc.VectorSubcoreMesh(core_axis_name="c", subcore_axis_name="s")
    f = functools.partial(
        pl.kernel,
        mesh=mesh,
        compiler_params=pltpu.CompilerParams(needs_layout_passes=False),
        out_type=jax.ShapeDtypeStruct((NUNITS, HW), jnp.float32),
        scratch_types=[
            pltpu.VMEM((HW,), jnp.float32),
            pltpu.VMEM((BLK,), jnp.float32),
            pltpu.VMEM((BLK,), jnp.float32),
            pltpu.VMEM((BLK,), jnp.float32),
            pltpu.VMEM((W,), jnp.float32),
        ],
    )(_warp_body)
    return f(img2, dxr, dyr)


def kernel(input_image, delta_x, delta_y):
    img2 = input_image.reshape(NUNITS, HW)
    dxr = delta_x.reshape(B, HW)
    dyr = delta_y.reshape(B, HW)
    out = _warp(img2, dxr, dyr)
    return out.reshape(B, C, H, W)


# speedup vs baseline: 7.0757x; 2.5771x over previous
import functools

import jax
import jax.numpy as jnp
from jax import lax
from jax.experimental import pallas as pl
from jax.experimental.pallas import tpu as pltpu
from jax.experimental.pallas import tpu_sc as plsc

B, C, H, W = 4, 192, 224, 224
HW = H * W
L = 16
ROWS_PER_BLK = 16
BLK = ROWS_PER_BLK * W
NBLK = H // ROWS_PER_BLK
NPAIR = NBLK // 2
VPB = BLK // L
VPR = W // L
NUNITS = B * C
NWORKERS = 32
UPW = NUNITS // NWORKERS
WPB = NWORKERS // B
SX = W / (W - 1.0)
SY = H / (H - 1.0)
DIV14_MUL, DIV14_SHIFT = 18725, 18


def _floor_to_int(v):
    t = v.astype(jnp.int32)
    tf = t.astype(jnp.float32)
    i0 = t - jnp.where(tf > v, 1, 0).astype(jnp.int32)
    return i0, i0.astype(jnp.float32)


def _warp_body(img_hbm, flow_hbm, out_hbm, imgbuf, fbufA, fbufB,
               outbufA, outbufB, xsbuf, sem_fA, sem_fB, sem_oA, sem_oB):
    wid = lax.axis_index("s") * 2 + lax.axis_index("c")
    b = wid // WPB

    def init_xs(i, _):
        xv = lax.iota(jnp.int32, L) + i * L
        xsbuf[pl.ds(i * L, L)] = xv.astype(jnp.float32) * SX - 0.5
        return 0
    lax.fori_loop(0, VPR, init_xs, 0)

    def compute_block(blk, fbuf, outbuf):
        @plsc.parallel_loop(0, VPB, unroll=4)
        def _(v):
            q = (v * DIV14_MUL) >> DIV14_SHIFT
            xv = v - q * VPR
            y = blk * ROWS_PER_BLK + q
            lv = v * L
            dxv = fbuf[0, pl.ds(lv, L)]
            dyv = fbuf[1, pl.ds(lv, L)]
            ix = xsbuf[pl.ds(xv * L, L)] + dxv * SX
            ys = y.astype(jnp.float32) * SY - 0.5
            iy = jnp.full((L,), ys, jnp.float32) + dyv * SY

            ix0, ix0f = _floor_to_int(ix)
            iy0, iy0f = _floor_to_int(iy)
            fx = ix - ix0f
            fy = iy - iy0f

            ix0c = jnp.minimum(jnp.maximum(ix0, 0), W - 1)
            ix1c = jnp.minimum(jnp.maximum(ix0 + 1, 0), W - 1)
            iy0c = jnp.minimum(jnp.maximum(iy0, 0), H - 1)
            iy1c = jnp.minimum(jnp.maximum(iy0 + 1, 0), H - 1)

            zero = jnp.zeros((L,), jnp.float32)
            wx0 = jnp.where((ix0f >= 0.0) & (ix0f <= W - 1.0), 1.0 - fx, zero)
            wx1 = jnp.where((ix0f >= -1.0) & (ix0f <= W - 2.0), fx, zero)
            wy0 = jnp.where((iy0f >= 0.0) & (iy0f <= H - 1.0), 1.0 - fy, zero)
            wy1 = jnp.where((iy0f >= -1.0) & (iy0f <= H - 2.0), fy, zero)

            row0 = iy0c * W
            row1 = iy1c * W
            v00 = plsc.load_gather(imgbuf, [row0 + ix0c])
            v01 = plsc.load_gather(imgbuf, [row0 + ix1c])
            v10 = plsc.load_gather(imgbuf, [row1 + ix0c])
            v11 = plsc.load_gather(imgbuf, [row1 + ix1c])

            acc = (v00 * wx0 + v01 * wx1) * wy0 + (v10 * wx0 + v11 * wx1) * wy1
            outbuf[pl.ds(lv, L)] = acc

    def flow_copy(blk, fbuf, sem):
        return pltpu.make_async_copy(
            flow_hbm.at[b, :, pl.ds(blk * BLK, BLK)], fbuf, sem)

    def out_copy(unit, blk, outbuf, sem):
        return pltpu.make_async_copy(
            outbuf, out_hbm.at[unit, pl.ds(blk * BLK, BLK)], sem)

    def unit_body(uu, _):
        unit = wid * UPW + uu
        flow_copy(0, fbufA, sem_fA).start()
        pltpu.sync_copy(img_hbm.at[unit], imgbuf)

        def pair_body(gg, _):
            a_blk = gg * 2
            b_blk = gg * 2 + 1
            flow_copy(b_blk, fbufB, sem_fB).start()
            flow_copy(a_blk, fbufA, sem_fA).wait()

            @pl.when(gg > 0)
            def _():
                out_copy(unit, a_blk - 2, outbufA, sem_oA).wait()
            compute_block(a_blk, fbufA, outbufA)
            out_copy(unit, a_blk, outbufA, sem_oA).start()

            @pl.when(gg < NPAIR - 1)
            def _():
                flow_copy(a_blk + 2, fbufA, sem_fA).start()
            flow_copy(b_blk, fbufB, sem_fB).wait()

            @pl.when(gg > 0)
            def _():
                out_copy(unit, b_blk - 2, outbufB, sem_oB).wait()
            compute_block(b_blk, fbufB, outbufB)
            out_copy(unit, b_blk, outbufB, sem_oB).start()
            return 0
        lax.fori_loop(0, NPAIR, pair_body, 0)
        out_copy(unit, NBLK - 2, outbufA, sem_oA).wait()
        out_copy(unit, NBLK - 1, outbufB, sem_oB).wait()
        return 0
    lax.fori_loop(0, UPW, unit_body, 0)


@jax.jit
def _warp(img2, flow):
    mesh = plsc.VectorSubcoreMesh(core_axis_name="c", subcore_axis_name="s")
    f = functools.partial(
        pl.kernel,
        mesh=mesh,
        compiler_params=pltpu.CompilerParams(needs_layout_passes=False),
        out_type=jax.ShapeDtypeStruct((NUNITS, HW), jnp.float32),
        scratch_types=[
            pltpu.VMEM((HW,), jnp.float32),
            pltpu.VMEM((2, BLK), jnp.float32),
            pltpu.VMEM((2, BLK), jnp.float32),
            pltpu.VMEM((BLK,), jnp.float32),
            pltpu.VMEM((BLK,), jnp.float32),
            pltpu.VMEM((W,), jnp.float32),
            pltpu.SemaphoreType.DMA,
            pltpu.SemaphoreType.DMA,
            pltpu.SemaphoreType.DMA,
            pltpu.SemaphoreType.DMA,
        ],
    )(_warp_body)
    return f(img2, flow)


def kernel(input_image, delta_x, delta_y):
    img2 = input_image.reshape(NUNITS, HW)
    flow = jnp.concatenate(
        [delta_x.reshape(B, 1, HW), delta_y.reshape(B, 1, HW)], axis=1)
    out = _warp(img2, flow)
    return out.reshape(B, C, H, W)


# speedup vs baseline: 7.0761x; 1.0001x over previous
import functools

import jax
import jax.numpy as jnp
from jax import lax
from jax.experimental import pallas as pl
from jax.experimental.pallas import tpu as pltpu
from jax.experimental.pallas import tpu_sc as plsc

B, C, H, W = 4, 192, 224, 224
HW = H * W
L = 16
ROWS_PER_BLK = 16
BLK = ROWS_PER_BLK * W
NBLK = H // ROWS_PER_BLK
NPAIR = NBLK // 2
VPB = BLK // L
VPR = W // L
NUNITS = B * C
NWORKERS = 32
UPW = NUNITS // NWORKERS
WPB = NWORKERS // B
SX = W / (W - 1.0)
SY = H / (H - 1.0)
DIV14_MUL, DIV14_SHIFT = 18725, 18


def _floor_to_int(v):
    t = v.astype(jnp.int32)
    tf = t.astype(jnp.float32)
    i0 = t - jnp.where(tf > v, 1, 0).astype(jnp.int32)
    return i0, i0.astype(jnp.float32)


def _warp_body(img_hbm, flow_hbm, out_hbm, imgbuf, fbufA, fbufB,
               outbufA, outbufB, xsbuf, sem_fA, sem_fB, sem_oA, sem_oB):
    wid = lax.axis_index("s") * 2 + lax.axis_index("c")
    b = wid // WPB

    def init_xs(i, _):
        xv = lax.iota(jnp.int32, L) + i * L
        xsbuf[pl.ds(i * L, L)] = xv.astype(jnp.float32) * SX - 0.5
        return 0
    lax.fori_loop(0, VPR, init_xs, 0)

    def compute_block(blk, fbuf, outbuf):
        @plsc.parallel_loop(0, VPB, unroll=4)
        def _(v):
            q = (v * DIV14_MUL) >> DIV14_SHIFT
            xv = v - q * VPR
            y = blk * ROWS_PER_BLK + q
            lv = v * L
            dxv = fbuf[0, pl.ds(lv, L)]
            dyv = fbuf[1, pl.ds(lv, L)]
            ix = xsbuf[pl.ds(xv * L, L)] + dxv * SX
            ys = y.astype(jnp.float32) * SY - 0.5
            iy = jnp.full((L,), ys, jnp.float32) + dyv * SY

            ix0, ix0f = _floor_to_int(ix)
            iy0, iy0f = _floor_to_int(iy)
            fx = ix - ix0f
            fy = iy - iy0f

            ix0c = jnp.minimum(jnp.maximum(ix0, 0), W - 1)
            ix1c = jnp.minimum(jnp.maximum(ix0 + 1, 0), W - 1)
            iy0c = jnp.minimum(jnp.maximum(iy0, 0), H - 1)
            iy1c = jnp.minimum(jnp.maximum(iy0 + 1, 0), H - 1)

            zero = jnp.zeros((L,), jnp.float32)
            wx0 = jnp.where((ix0f >= 0.0) & (ix0f <= W - 1.0), 1.0 - fx, zero)
            wx1 = jnp.where((ix0f >= -1.0) & (ix0f <= W - 2.0), fx, zero)
            wy0 = jnp.where((iy0f >= 0.0) & (iy0f <= H - 1.0), 1.0 - fy, zero)
            wy1 = jnp.where((iy0f >= -1.0) & (iy0f <= H - 2.0), fy, zero)

            row0 = iy0c * W
            row1 = iy1c * W
            v00 = plsc.load_gather(imgbuf, [row0 + ix0c])
            v01 = plsc.load_gather(imgbuf, [row0 + ix1c])
            v10 = plsc.load_gather(imgbuf, [row1 + ix0c])
            v11 = plsc.load_gather(imgbuf, [row1 + ix1c])

            acc = (v00 * wx0 + v01 * wx1) * wy0 + (v10 * wx0 + v11 * wx1) * wy1
            outbuf[pl.ds(lv, L)] = acc

    def flow_copy(blk, fbuf, sem):
        return pltpu.make_async_copy(
            flow_hbm.at[b, :, pl.ds(blk * BLK, BLK)], fbuf, sem)

    def out_copy(unit, blk, outbuf, sem):
        return pltpu.make_async_copy(
            outbuf, out_hbm.at[unit, pl.ds(blk * BLK, BLK)], sem)

    def unit_body(uu, _):
        unit = wid * UPW + uu
        flow_copy(0, fbufA, sem_fA).start()
        pltpu.sync_copy(img_hbm.at[unit], imgbuf)

        def pair_body(gg, _):
            a_blk = gg * 2
            b_blk = gg * 2 + 1
            flow_copy(b_blk, fbufB, sem_fB).start()
            flow_copy(a_blk, fbufA, sem_fA).wait()

            @pl.when(gg > 0)
            def _():
                out_copy(unit, a_blk - 2, outbufA, sem_oA).wait()
            compute_block(a_blk, fbufA, outbufA)
            out_copy(unit, a_blk, outbufA, sem_oA).start()

            @pl.when(gg < NPAIR - 1)
            def _():
                flow_copy(a_blk + 2, fbufA, sem_fA).start()
            flow_copy(b_blk, fbufB, sem_fB).wait()

            @pl.when(gg > 0)
            def _():
                out_copy(unit, b_blk - 2, outbufB, sem_oB).wait()
            compute_block(b_blk, fbufB, outbufB)
            out_copy(unit, b_blk, outbufB, sem_oB).start()
            return 0
        lax.fori_loop(0, NPAIR, pair_body, 0)
        out_copy(unit, NBLK - 2, outbufA, sem_oA).wait()
        out_copy(unit, NBLK - 1, outbufB, sem_oB).wait()
        return 0
    lax.fori_loop(0, UPW, unit_body, 0)


@jax.jit
def _warp(img2, flow):
    mesh = plsc.VectorSubcoreMesh(core_axis_name="c", subcore_axis_name="s")
    f = functools.partial(
        pl.kernel,
        mesh=mesh,
        compiler_params=pltpu.CompilerParams(
            needs_layout_passes=False, use_tc_tiling_on_sc=True),
        out_type=jax.ShapeDtypeStruct((NUNITS, HW), jnp.float32),
        scratch_types=[
            pltpu.VMEM((HW,), jnp.float32),
            pltpu.VMEM((2, BLK), jnp.float32),
            pltpu.VMEM((2, BLK), jnp.float32),
            pltpu.VMEM((BLK,), jnp.float32),
            pltpu.VMEM((BLK,), jnp.float32),
            pltpu.VMEM((W,), jnp.float32),
            pltpu.SemaphoreType.DMA,
            pltpu.SemaphoreType.DMA,
            pltpu.SemaphoreType.DMA,
            pltpu.SemaphoreType.DMA,
        ],
    )(_warp_body)
    return f(img2, flow)


def kernel(input_image, delta_x, delta_y):
    img2 = input_image.reshape(NUNITS, HW)
    flow = jnp.concatenate(
        [delta_x.reshape(B, 1, HW), delta_y.reshape(B, 1, HW)], axis=1)
    out = _warp(img2, flow)
    return out.reshape(B, C, H, W)
